# Initial kernel scaffold; baseline (speedup 1.0000x reference)
import jax
import jax.numpy as jnp
from jax.experimental import pallas as pl


def kernel(x, W1l, b1l, W1r, W2l, b2l, W2r, Wa, ba, Wb, bb, edge_index, pos_edge_index, neg_edge_index):
    raise NotImplementedError("write your pallas kernel here")


# speedup vs baseline: 2.7614x; 2.7614x over previous
import functools

import jax
import jax.numpy as jnp
from jax import lax
from jax.experimental import pallas as pl
from jax.experimental.pallas import tpu as pltpu
from jax.experimental.pallas import tpu_sc as plsc

N = 10000
E = 320000
PE = 100000
D = 128

NC = 2
NS = 16
NW = NC * NS

C = 128
EP = 327680
PEP = 102400
EP2 = 2 * PEP
NP = 10240
RPW = NP // NS

_mesh = lambda: plsc.VectorSubcoreMesh(core_axis_name="c", subcore_axis_name="s", num_cores=NC, num_subcores=NS)


def _make_spmm(with_deg):
  out_type = [jax.ShapeDtypeStruct((NC * NP, D), jnp.float32)]
  if with_deg:
    out_type.append(jax.ShapeDtypeStruct((NC * NP,), jnp.float32))
  scratch = [
      pltpu.VMEM((C,), jnp.int32),
      pltpu.VMEM((C,), jnp.int32),
      pltpu.VMEM((C, D), jnp.float32),
      pltpu.VMEM_SHARED((NP, D), jnp.float32),
      pltpu.SemaphoreType.DMA,
  ]
  if with_deg:
    scratch += [
        pltpu.VMEM((NP,), jnp.float32),
        pltpu.VMEM((RPW,), jnp.float32),
        pltpu.VMEM((RPW,), jnp.float32),
        pltpu.VMEM_SHARED((NS * NP,), jnp.float32),
    ]

  @functools.partial(pl.kernel, out_type=out_type, mesh=_mesh(),
                     scratch_types=scratch,
                     compiler_params=pltpu.CompilerParams(
                         needs_layout_passes=False))
  def spmm(*refs):
    if with_deg:
      (table_h, src_h, dst_h, out_h, deg_h,
       src_v, dst_v, rows_v, acc_sh, sem,
       hist_v, sum_v, tmp_v, hist_sh) = refs
    else:
      (table_h, src_h, dst_h, out_h,
       src_v, dst_v, rows_v, acc_sh, sem) = refs
    cid = lax.axis_index("c")
    sid = lax.axis_index("s")
    wid = sid * NC + cid

    zero16 = jnp.zeros((16,), jnp.float32)
    def zrow(i, _):
      for j in range(D // 16):
        rows_v[i, pl.ds(16 * j, 16)] = zero16
      return 0
    lax.fori_loop(0, C, zrow, 0)
    if with_deg:
      def zhist(i, _):
        hist_v[pl.ds(i * 16, 16)] = zero16
        return 0
      lax.fori_loop(0, NP // 16, zhist, 0)

    row0 = sid * RPW
    def zacc(m, _):
      pltpu.sync_copy(rows_v, acc_sh.at[pl.ds(row0 + m * C, C)])
      return 0
    lax.fori_loop(0, RPW // C, zacc, 0)

    plsc.subcore_barrier()

    nch = EP // (NW * C)
    base0 = wid * nch * C
    def step(k, _):
      b = base0 + k * C
      pltpu.sync_copy(src_h.at[pl.ds(b, C)], src_v)
      pltpu.sync_copy(dst_h.at[pl.ds(b, C)], dst_v)
      pltpu.async_copy(table_h.at[src_v], rows_v, sem).wait()
      pltpu.sync_copy(rows_v, acc_sh.at[dst_v], add=True)
      if with_deg:
        for q in range(C // 16):
          idx16 = dst_v[pl.ds(q * 16, 16)]
          cnt, last = plsc.scan_count(idx16)
          plsc.addupdate_scatter(hist_v, [idx16], cnt.astype(jnp.float32),
                                 mask=last)
      return 0
    lax.fori_loop(0, nch, step, 0)

    plsc.subcore_barrier()

    def cp(m, _):
      r = row0 + m * C
      pltpu.sync_copy(acc_sh.at[pl.ds(r, C)], out_h.at[pl.ds(cid * NP + r, C)])
      return 0
    lax.fori_loop(0, RPW // C, cp, 0)

    if with_deg:
      pltpu.sync_copy(hist_v, hist_sh.at[pl.ds(sid * NP, NP)])
      plsc.subcore_barrier()
      pltpu.sync_copy(hist_sh.at[pl.ds(row0, RPW)], sum_v)
      def red(t, _):
        pltpu.sync_copy(hist_sh.at[pl.ds(t * NP + row0, RPW)], tmp_v)
        def addv(m, _):
          sum_v[pl.ds(m * 16, 16)] = (sum_v[pl.ds(m * 16, 16)]
                                      + tmp_v[pl.ds(m * 16, 16)])
          return 0
        lax.fori_loop(0, RPW // 16, addv, 0)
        return 0
      lax.fori_loop(1, NS, red, 0)
      pltpu.sync_copy(sum_v, deg_h.at[pl.ds(cid * NP + row0, RPW)])

  return spmm


_spmm_deg = _make_spmm(True)
_spmm = _make_spmm(False)


def _tc_layer1(parts, deg, x, wlT, bl, wrT):
  R = 400
  def body(parts_ref, deg_ref, x_ref, wlT_ref, bl_ref, wrT_ref, out_ref):
    aggsum = parts_ref[0] + parts_ref[1]
    d = deg_ref[0] + deg_ref[1]
    recip = 1.0 / jnp.maximum(d, 1.0)
    y = (jnp.dot(aggsum, wlT_ref[...], preferred_element_type=jnp.float32)
         * recip + bl_ref[...]
         + jnp.dot(x_ref[...], wrT_ref[...], preferred_element_type=jnp.float32))
    out_ref[...] = jnp.maximum(y, 0.0)
  return pl.pallas_call(
      body,
      grid=(N // R,),
      in_specs=[
          pl.BlockSpec((2, R, D), lambda i: (0, i, 0)),
          pl.BlockSpec((2, R, 1), lambda i: (0, i, 0)),
          pl.BlockSpec((R, D), lambda i: (i, 0)),
          pl.BlockSpec((D, D), lambda i: (0, 0)),
          pl.BlockSpec((1, D), lambda i: (0, 0)),
          pl.BlockSpec((D, D), lambda i: (0, 0)),
      ],
      out_specs=pl.BlockSpec((R, D), lambda i: (i, 0)),
      out_shape=jax.ShapeDtypeStruct((N, D), jnp.float32),
  )(parts, deg, x, wlT, bl, wrT)


def _tc_layer2(parts, deg, z1, wlT, bl, wrT, walT, ba, warT):
  R = 400
  def body(parts_ref, deg_ref, z1_ref, wlT_ref, bl_ref, wrT_ref,
           walT_ref, ba_ref, warT_ref, u_ref, v_ref):
    aggsum = parts_ref[0] + parts_ref[1]
    d = deg_ref[0] + deg_ref[1]
    recip = 1.0 / jnp.maximum(d, 1.0)
    z2 = (jnp.dot(aggsum, wlT_ref[...], preferred_element_type=jnp.float32)
          * recip + bl_ref[...]
          + jnp.dot(z1_ref[...], wrT_ref[...], preferred_element_type=jnp.float32))
    u_ref[...] = jnp.dot(z2, walT_ref[...],
                         preferred_element_type=jnp.float32) + ba_ref[...]
    v_ref[...] = jnp.dot(z2, warT_ref[...],
                         preferred_element_type=jnp.float32)
  return pl.pallas_call(
      body,
      grid=(N // R,),
      in_specs=[
          pl.BlockSpec((2, R, D), lambda i: (0, i, 0)),
          pl.BlockSpec((2, R, 1), lambda i: (0, i, 0)),
          pl.BlockSpec((R, D), lambda i: (i, 0)),
          pl.BlockSpec((D, D), lambda i: (0, 0)),
          pl.BlockSpec((1, D), lambda i: (0, 0)),
          pl.BlockSpec((D, D), lambda i: (0, 0)),
          pl.BlockSpec((D, D), lambda i: (0, 0)),
          pl.BlockSpec((1, D), lambda i: (0, 0)),
          pl.BlockSpec((D, D), lambda i: (0, 0)),
      ],
      out_specs=[pl.BlockSpec((R, D), lambda i: (i, 0)),
                 pl.BlockSpec((R, D), lambda i: (i, 0))],
      out_shape=[jax.ShapeDtypeStruct((N, D), jnp.float32),
                 jax.ShapeDtypeStruct((N, D), jnp.float32)],
  )(parts, deg, z1, wlT, bl, wrT, walT, ba, warT)


@functools.partial(
    pl.kernel,
    out_type=jax.ShapeDtypeStruct((EP2,), jnp.float32),
    mesh=_mesh(),
    scratch_types=[
        pltpu.VMEM((C,), jnp.int32),
        pltpu.VMEM((C,), jnp.int32),
        pltpu.VMEM((C, D), jnp.float32),
        pltpu.VMEM((C, D), jnp.float32),
        pltpu.VMEM((C * 16,), jnp.float32),
        pltpu.VMEM((C,), jnp.float32),
        pltpu.VMEM((D + 16,), jnp.float32),
        pltpu.SemaphoreType.DMA,
        pltpu.SemaphoreType.DMA,
    ],
    compiler_params=pltpu.CompilerParams(needs_layout_passes=False),
)
def _decode(u_h, v_h, src_h, dst_h, wbb_h, out_h,
            src_v, dst_v, u_rows, v_rows, sums_v, pred_v, wbb_v, sem_u, sem_v):
  cid = lax.axis_index("c")
  sid = lax.axis_index("s")
  wid = sid * NC + cid
  pltpu.sync_copy(wbb_h, wbb_v)
  wbs = [wbb_v[pl.ds(16 * j, 16)] for j in range(D // 16)]
  bbv = wbb_v[pl.ds(D, 16)]

  nch = EP2 // (NW * C)
  base0 = wid * nch * C
  def step(k, _):
    b = base0 + k * C
    pltpu.sync_copy(src_h.at[pl.ds(b, C)], src_v)
    pltpu.sync_copy(dst_h.at[pl.ds(b, C)], dst_v)
    cu = pltpu.async_copy(u_h.at[src_v], u_rows, sem_u)
    cv = pltpu.async_copy(v_h.at[dst_v], v_rows, sem_v)
    cu.wait()
    cv.wait()
    lanes = lax.iota(jnp.int32, 16)
    def group(g, _):
      out_vec = jnp.zeros((16,), jnp.float32)
      for i in range(16):
        e = g * 16 + i
        acc = bbv
        for j in range(D // 16):
          t = jnp.maximum(u_rows[e, pl.ds(16 * j, 16)]
                          + v_rows[e, pl.ds(16 * j, 16)], 0.0)
          acc = acc + t * wbs[j]
        out_vec = jnp.where(lanes == i, jnp.sum(acc), out_vec)
      pred_v[pl.ds(g * 16, 16)] = out_vec
      return 0
    lax.fori_loop(0, C // 16, group, 0)
    pltpu.sync_copy(pred_v, out_h.at[pl.ds(b, C)])
    return 0
  lax.fori_loop(0, nch, step, 0)


def kernel(x, W1l, b1l, W1r, W2l, b2l, W2r, Wa, ba, Wb, bb,
           edge_index, pos_edge_index, neg_edge_index):
  i32 = jnp.int32
  f32 = jnp.float32
  src = edge_index[0].astype(i32)
  dst = edge_index[1].astype(i32)
  pad_e = EP - E
  src_p = jnp.concatenate([src, jnp.zeros((pad_e,), i32)])
  dst_p = jnp.concatenate([dst, jnp.full((pad_e,), N, i32)])

  parts1, degp = _spmm_deg(x, src_p, dst_p)
  parts1 = parts1.reshape(NC, NP, D)
  degp = degp.reshape(NC, NP, 1)
  z1 = _tc_layer1(parts1, degp, x, W1l.T, b1l.reshape(1, D), W1r.T)

  parts2, = _spmm(z1, src_p, dst_p)
  parts2 = parts2.reshape(NC, NP, D)
  U, V = _tc_layer2(parts2, degp, z1, W2l.T, b2l.reshape(1, D), W2r.T,
                    Wa[:, :D].T, ba.reshape(1, D), Wa[:, D:].T)

  padp = PEP - PE
  zi = jnp.zeros((padp,), i32)
  src_all = jnp.concatenate([pos_edge_index[0].astype(i32), zi,
                             neg_edge_index[0].astype(i32), zi])
  dst_all = jnp.concatenate([pos_edge_index[1].astype(i32), zi,
                             neg_edge_index[1].astype(i32), zi])
  wbb = jnp.concatenate([Wb[0], bb, jnp.zeros((15,), f32)])

  preds = _decode(U, V, src_all, dst_all, wbb)
  return preds[:PE], preds[PEP:PEP + PE]


# speedup vs baseline: 2.8650x; 1.0375x over previous
import functools

import jax
import jax.numpy as jnp
from jax import lax
from jax.experimental import pallas as pl
from jax.experimental.pallas import tpu as pltpu
from jax.experimental.pallas import tpu_sc as plsc

N = 10000
E = 320000
PE = 100000
D = 128

NC = 2
NS = 16
NW = NC * NS

C = 128
EP = 327680
PEP = 102400
EP2 = 2 * PEP
NP = 10240
RPW = NP // NS

_mesh = lambda: plsc.VectorSubcoreMesh(core_axis_name="c", subcore_axis_name="s", num_cores=NC, num_subcores=NS)


def _make_spmm(with_deg):
  out_type = [jax.ShapeDtypeStruct((NC * NP, D), jnp.float32)]
  if with_deg:
    out_type.append(jax.ShapeDtypeStruct((NW * NP,), jnp.float32))
  scratch = [
      pltpu.VMEM((C,), jnp.int32),
      pltpu.VMEM((C,), jnp.int32),
      pltpu.VMEM((C, D), jnp.float32),
      pltpu.VMEM((C,), jnp.int32),
      pltpu.VMEM((C,), jnp.int32),
      pltpu.VMEM((C, D), jnp.float32),
      pltpu.VMEM_SHARED((NP, D), jnp.float32),
      pltpu.SemaphoreType.DMA,
      pltpu.SemaphoreType.DMA,
  ]
  if with_deg:
    scratch += [
        pltpu.VMEM((NP,), jnp.float32),
    ]

  @functools.partial(pl.kernel, out_type=out_type, mesh=_mesh(),
                     scratch_types=scratch,
                     compiler_params=pltpu.CompilerParams(
                         needs_layout_passes=False))
  def spmm(*refs):
    if with_deg:
      (table_h, src_h, dst_h, out_h, deg_h,
       src_v, dst_v, rows_v, srcb_v, dstb_v, rowsb_v, acc_sh, sem, semb,
       hist_v) = refs
    else:
      (table_h, src_h, dst_h, out_h,
       src_v, dst_v, rows_v, srcb_v, dstb_v, rowsb_v, acc_sh, sem, semb) = refs
    cid = lax.axis_index("c")
    sid = lax.axis_index("s")
    wid = sid * NC + cid

    zero16 = jnp.zeros((16,), jnp.float32)
    def zrow(i, _):
      for j in range(D // 16):
        rows_v[i, pl.ds(16 * j, 16)] = zero16
      return 0
    lax.fori_loop(0, C, zrow, 0)
    if with_deg:
      def zhist(i, _):
        hist_v[pl.ds(i * 16, 16)] = zero16
        return 0
      lax.fori_loop(0, NP // 16, zhist, 0)

    row0 = sid * RPW
    def zacc(m, _):
      pltpu.sync_copy(rows_v, acc_sh.at[pl.ds(row0 + m * C, C)])
      return 0
    lax.fori_loop(0, RPW // C, zacc, 0)

    plsc.subcore_barrier()

    nch = EP // (NW * C)
    base0 = wid * nch * C

    def hist_update(dref):
      if with_deg:
        for q in range(C // 16):
          idx16 = dref[pl.ds(q * 16, 16)]
          cnt, last = plsc.scan_count(idx16)
          plsc.addupdate_scatter(hist_v, [idx16], cnt.astype(jnp.float32),
                                 mask=last)

    def wait_gather(buf, s):
      pltpu.make_async_copy(table_h.at[pl.ds(0, C)], buf, s).wait()

    pltpu.sync_copy(src_h.at[pl.ds(base0, C)], src_v)
    pltpu.sync_copy(dst_h.at[pl.ds(base0, C)], dst_v)
    pltpu.async_copy(table_h.at[src_v], rows_v, sem)

    def step2(k2, _):
      b = base0 + 2 * k2 * C
      pltpu.sync_copy(src_h.at[pl.ds(b + C, C)], srcb_v)
      pltpu.sync_copy(dst_h.at[pl.ds(b + C, C)], dstb_v)
      pltpu.async_copy(table_h.at[srcb_v], rowsb_v, semb)
      hist_update(dst_v)
      wait_gather(rows_v, sem)
      pltpu.sync_copy(rows_v, acc_sh.at[dst_v], add=True)
      @pl.when(k2 < nch // 2 - 1)
      def _():
        pltpu.sync_copy(src_h.at[pl.ds(b + 2 * C, C)], src_v)
        pltpu.sync_copy(dst_h.at[pl.ds(b + 2 * C, C)], dst_v)
        pltpu.async_copy(table_h.at[src_v], rows_v, sem)
      hist_update(dstb_v)
      wait_gather(rowsb_v, semb)
      pltpu.sync_copy(rowsb_v, acc_sh.at[dstb_v], add=True)
      return 0
    lax.fori_loop(0, nch // 2, step2, 0)

    plsc.subcore_barrier()

    def cp(m, _):
      r = row0 + m * C
      pltpu.sync_copy(acc_sh.at[pl.ds(r, C)], out_h.at[pl.ds(cid * NP + r, C)])
      return 0
    lax.fori_loop(0, RPW // C, cp, 0)

    if with_deg:
      pltpu.sync_copy(hist_v, deg_h.at[pl.ds(wid * NP, NP)])

  return spmm


_spmm_deg = _make_spmm(True)
_spmm = _make_spmm(False)


def _tc_layer1(parts, deg, x, wlT, bl, wrT):
  R = 400
  def body(parts_ref, deg_ref, x_ref, wlT_ref, bl_ref, wrT_ref, out_ref):
    aggsum = parts_ref[0] + parts_ref[1]
    d = jnp.sum(deg_ref[...], axis=0)
    recip = 1.0 / jnp.maximum(d, 1.0)
    y = (jnp.dot(aggsum, wlT_ref[...], preferred_element_type=jnp.float32)
         * recip + bl_ref[...]
         + jnp.dot(x_ref[...], wrT_ref[...], preferred_element_type=jnp.float32))
    out_ref[...] = jnp.maximum(y, 0.0)
  return pl.pallas_call(
      body,
      grid=(N // R,),
      in_specs=[
          pl.BlockSpec((2, R, D), lambda i: (0, i, 0)),
          pl.BlockSpec((NW, R, 1), lambda i: (0, i, 0)),
          pl.BlockSpec((R, D), lambda i: (i, 0)),
          pl.BlockSpec((D, D), lambda i: (0, 0)),
          pl.BlockSpec((1, D), lambda i: (0, 0)),
          pl.BlockSpec((D, D), lambda i: (0, 0)),
      ],
      out_specs=pl.BlockSpec((R, D), lambda i: (i, 0)),
      out_shape=jax.ShapeDtypeStruct((N, D), jnp.float32),
  )(parts, deg, x, wlT, bl, wrT)


def _tc_layer2(parts, deg, z1, wlT, bl, wrT, walT, ba, warT):
  R = 400
  def body(parts_ref, deg_ref, z1_ref, wlT_ref, bl_ref, wrT_ref,
           walT_ref, ba_ref, warT_ref, u_ref, v_ref):
    aggsum = parts_ref[0] + parts_ref[1]
    d = jnp.sum(deg_ref[...], axis=0)
    recip = 1.0 / jnp.maximum(d, 1.0)
    z2 = (jnp.dot(aggsum, wlT_ref[...], preferred_element_type=jnp.float32)
          * recip + bl_ref[...]
          + jnp.dot(z1_ref[...], wrT_ref[...], preferred_element_type=jnp.float32))
    u_ref[...] = jnp.dot(z2, walT_ref[...],
                         preferred_element_type=jnp.float32) + ba_ref[...]
    v_ref[...] = jnp.dot(z2, warT_ref[...],
                         preferred_element_type=jnp.float32)
  return pl.pallas_call(
      body,
      grid=(N // R,),
      in_specs=[
          pl.BlockSpec((2, R, D), lambda i: (0, i, 0)),
          pl.BlockSpec((NW, R, 1), lambda i: (0, i, 0)),
          pl.BlockSpec((R, D), lambda i: (i, 0)),
          pl.BlockSpec((D, D), lambda i: (0, 0)),
          pl.BlockSpec((1, D), lambda i: (0, 0)),
          pl.BlockSpec((D, D), lambda i: (0, 0)),
          pl.BlockSpec((D, D), lambda i: (0, 0)),
          pl.BlockSpec((1, D), lambda i: (0, 0)),
          pl.BlockSpec((D, D), lambda i: (0, 0)),
      ],
      out_specs=[pl.BlockSpec((R, D), lambda i: (i, 0)),
                 pl.BlockSpec((R, D), lambda i: (i, 0))],
      out_shape=[jax.ShapeDtypeStruct((N, D), jnp.float32),
                 jax.ShapeDtypeStruct((N, D), jnp.float32)],
  )(parts, deg, z1, wlT, bl, wrT, walT, ba, warT)


@functools.partial(
    pl.kernel,
    out_type=jax.ShapeDtypeStruct((EP2,), jnp.float32),
    mesh=_mesh(),
    scratch_types=[
        pltpu.VMEM((C,), jnp.int32),
        pltpu.VMEM((C,), jnp.int32),
        pltpu.VMEM((C, D), jnp.float32),
        pltpu.VMEM((C, D), jnp.float32),
        pltpu.VMEM((C,), jnp.float32),
        pltpu.VMEM((C,), jnp.int32),
        pltpu.VMEM((C,), jnp.int32),
        pltpu.VMEM((C, D), jnp.float32),
        pltpu.VMEM((C, D), jnp.float32),
        pltpu.VMEM((C,), jnp.float32),
        pltpu.VMEM((D + 16,), jnp.float32),
        pltpu.SemaphoreType.DMA,
        pltpu.SemaphoreType.DMA,
    ],
    compiler_params=pltpu.CompilerParams(needs_layout_passes=False),
)
def _decode(u_h, v_h, src_h, dst_h, wbb_h, out_h,
            src_v, dst_v, u_rows, v_rows, pred_v,
            srcb_v, dstb_v, u_rowsb, v_rowsb, predb_v,
            wbb_v, sem_a, sem_b):
  cid = lax.axis_index("c")
  sid = lax.axis_index("s")
  wid = sid * NC + cid
  pltpu.sync_copy(wbb_h, wbb_v)
  wbs = [wbb_v[pl.ds(16 * j, 16)] for j in range(D // 16)]
  bbv = wbb_v[pl.ds(D, 16)]
  lanes = lax.iota(jnp.int32, 16)

  def compute(ur, vr, pr):
    def group(g, _):
      out_vec = jnp.zeros((16,), jnp.float32)
      for i in range(16):
        e = g * 16 + i
        acc = bbv
        for j in range(D // 16):
          t = jnp.maximum(ur[e, pl.ds(16 * j, 16)]
                          + vr[e, pl.ds(16 * j, 16)], 0.0)
          acc = acc + t * wbs[j]
        out_vec = jnp.where(lanes == i, jnp.sum(acc), out_vec)
      pr[pl.ds(g * 16, 16)] = out_vec
      return 0
    lax.fori_loop(0, C // 16, group, 0)

  def wait_pair(ur, vr, s):
    pltpu.make_async_copy(u_h.at[pl.ds(0, C)], ur, s).wait()
    pltpu.make_async_copy(u_h.at[pl.ds(0, C)], vr, s).wait()

  nch = EP2 // (NW * C)
  base0 = wid * nch * C
  pltpu.sync_copy(src_h.at[pl.ds(base0, C)], src_v)
  pltpu.sync_copy(dst_h.at[pl.ds(base0, C)], dst_v)
  pltpu.async_copy(u_h.at[src_v], u_rows, sem_a)
  pltpu.async_copy(v_h.at[dst_v], v_rows, sem_a)

  def step2(k2, _):
    b = base0 + 2 * k2 * C
    pltpu.sync_copy(src_h.at[pl.ds(b + C, C)], srcb_v)
    pltpu.sync_copy(dst_h.at[pl.ds(b + C, C)], dstb_v)
    pltpu.async_copy(u_h.at[srcb_v], u_rowsb, sem_b)
    pltpu.async_copy(v_h.at[dstb_v], v_rowsb, sem_b)
    wait_pair(u_rows, v_rows, sem_a)
    compute(u_rows, v_rows, pred_v)
    pltpu.sync_copy(pred_v, out_h.at[pl.ds(b, C)])
    @pl.when(k2 < nch // 2 - 1)
    def _():
      pltpu.sync_copy(src_h.at[pl.ds(b + 2 * C, C)], src_v)
      pltpu.sync_copy(dst_h.at[pl.ds(b + 2 * C, C)], dst_v)
      pltpu.async_copy(u_h.at[src_v], u_rows, sem_a)
      pltpu.async_copy(v_h.at[dst_v], v_rows, sem_a)
    wait_pair(u_rowsb, v_rowsb, sem_b)
    compute(u_rowsb, v_rowsb, predb_v)
    pltpu.sync_copy(predb_v, out_h.at[pl.ds(b + C, C)])
    return 0
  lax.fori_loop(0, nch // 2, step2, 0)


def kernel(x, W1l, b1l, W1r, W2l, b2l, W2r, Wa, ba, Wb, bb,
           edge_index, pos_edge_index, neg_edge_index):
  i32 = jnp.int32
  f32 = jnp.float32
  src = edge_index[0].astype(i32)
  dst = edge_index[1].astype(i32)
  pad_e = EP - E
  src_p = jnp.concatenate([src, jnp.zeros((pad_e,), i32)])
  dst_p = jnp.concatenate([dst, jnp.full((pad_e,), N, i32)])

  parts1, degp = _spmm_deg(x, src_p, dst_p)
  parts1 = parts1.reshape(NC, NP, D)
  degp = degp.reshape(NW, NP, 1)
  z1 = _tc_layer1(parts1, degp, x, W1l.T, b1l.reshape(1, D), W1r.T)

  parts2, = _spmm(z1, src_p, dst_p)
  parts2 = parts2.reshape(NC, NP, D)
  U, V = _tc_layer2(parts2, degp, z1, W2l.T, b2l.reshape(1, D), W2r.T,
                    Wa[:, :D].T, ba.reshape(1, D), Wa[:, D:].T)

  padp = PEP - PE
  zi = jnp.zeros((padp,), i32)
  src_all = jnp.concatenate([pos_edge_index[0].astype(i32), zi,
                             neg_edge_index[0].astype(i32), zi])
  dst_all = jnp.concatenate([pos_edge_index[1].astype(i32), zi,
                             neg_edge_index[1].astype(i32), zi])
  wbb = jnp.concatenate([Wb[0], bb, jnp.zeros((15,), f32)])

  preds = _decode(U, V, src_all, dst_all, wbb)
  return preds[:PE], preds[PEP:PEP + PE]
